# baseline (speedup 1.0000x reference)
import functools

import jax
import jax.numpy as jnp
from jax import lax
from jax.experimental import pallas as pl
from jax.experimental.pallas import tpu as pltpu
from jax.experimental.pallas import tpu_sc as plsc

BATCH = 16384
HIST = 50
DEPTH = 101
ROW_W = HIST * DEPTH

NC, NS, L = 2, 16, 16
NW = NC * NS
ROWS_PER_W = BATCH // NW
R = 8
NIDX = R * HIST
NVEC = NIDX // L
BUF_W = R * ROW_W
ITERS = ROWS_PER_W // R
IDX_PER_W = ROWS_PER_W * HIST


def _body(in_hbm, out_hbm, idx_v, base_v, buf0, buf1, offs0, offs1, sem0, sem1):
    wid = lax.axis_index("s") * NC + lax.axis_index("c")

    pltpu.sync_copy(in_hbm.at[pl.ds(wid * IDX_PER_W, IDX_PER_W)], idx_v)

    zeros_f = jnp.zeros((L,), jnp.float32)
    ones_f = jnp.ones((L,), jnp.float32)

    def _zero(k, _):
        buf0[pl.ds(k * L, L)] = zeros_f
        buf1[pl.ds(k * L, L)] = zeros_f
        return _

    lax.fori_loop(0, BUF_W // L, _zero, None)

    lane = lax.iota(jnp.int32, L)
    for v in range(NVEC):
        t = lane + v * L
        row = t // HIST
        j = t - row * HIST
        base_v[pl.ds(v * L, L)] = row * ROW_W + j * DEPTH

    out_base = wid * ROWS_PER_W * ROW_W

    def _restore(buf, offs):
        for v in range(NVEC):
            off = offs[pl.ds(v * L, L)]
            plsc.store_scatter(buf, [off], zeros_f)

    def _fill_and_fire(it, buf, offs, sem):
        for v in range(NVEC):
            idxv = idx_v[pl.ds(it * NIDX + v * L, L)]
            off = base_v[pl.ds(v * L, L)] + idxv
            offs[pl.ds(v * L, L)] = off
            plsc.store_scatter(buf, [off], ones_f)
        pltpu.async_copy(buf, out_hbm.at[pl.ds(out_base + it * BUF_W, BUF_W)], sem)

    _fill_and_fire(0, buf0, offs0, sem0)
    _fill_and_fire(1, buf1, offs1, sem1)

    def _pair(p, _):
        it0 = 2 * p
        pltpu.make_async_copy(buf0, out_hbm.at[pl.ds(out_base, BUF_W)], sem0).wait()
        _restore(buf0, offs0)
        _fill_and_fire(it0, buf0, offs0, sem0)
        pltpu.make_async_copy(buf1, out_hbm.at[pl.ds(out_base, BUF_W)], sem1).wait()
        _restore(buf1, offs1)
        _fill_and_fire(it0 + 1, buf1, offs1, sem1)
        return _

    lax.fori_loop(1, ITERS // 2, _pair, None)

    pltpu.make_async_copy(buf0, out_hbm.at[pl.ds(out_base, BUF_W)], sem0).wait()
    pltpu.make_async_copy(buf1, out_hbm.at[pl.ds(out_base, BUF_W)], sem1).wait()


@jax.jit
def _one_hot_sc(flat_idx):
    mesh = plsc.VectorSubcoreMesh(core_axis_name="c", subcore_axis_name="s")
    return pl.kernel(
        _body,
        out_type=jax.ShapeDtypeStruct((BATCH * ROW_W,), jnp.float32),
        mesh=mesh,
        scratch_types=[
            pltpu.VMEM((IDX_PER_W,), jnp.int32),
            pltpu.VMEM((NIDX,), jnp.int32),
            pltpu.VMEM((BUF_W,), jnp.float32),
            pltpu.VMEM((BUF_W,), jnp.float32),
            pltpu.VMEM((NIDX,), jnp.int32),
            pltpu.VMEM((NIDX,), jnp.int32),
            pltpu.SemaphoreType.DMA,
            pltpu.SemaphoreType.DMA,
        ],
        compiler_params=pltpu.CompilerParams(needs_layout_passes=False),
    )(flat_idx)


def kernel(inputs):
    flat = inputs.reshape(BATCH * HIST)
    out = _one_hot_sc(flat)
    return out.reshape(BATCH, HIST, DEPTH)


# speedup vs baseline: 7.4083x; 7.4083x over previous
import jax
import jax.numpy as jnp
from jax import lax
from jax.experimental import pallas as pl
from jax.experimental.pallas import tpu as pltpu
from jax.experimental.pallas import tpu_sc as plsc

BATCH = 16384
HIST = 50
DEPTH = 101
L = 16
NC, NS = 2, 16
NW = NC * NS
BPW = BATCH // NW
SUB = 256
NSUB = BPW // SUB
NVEC = SUB // L
IDX_PER_W = BPW * HIST


def _body(in_hbm, out_hbm, idx_v, basepos, buf0, buf1, offs0, offs1, sem0, sem1):
    wid = lax.axis_index("s") * NC + lax.axis_index("c")
    b0 = wid * BPW

    pltpu.sync_copy(in_hbm.at[pl.ds(b0 * HIST, IDX_PER_W)], idx_v)

    zeros_f = jnp.zeros((L,), jnp.float32)
    ones_f = jnp.ones((L,), jnp.float32)
    lane = lax.iota(jnp.int32, L)

    for v in range(BPW // L):
        basepos[pl.ds(v * L, L)] = (lane + v * L) * HIST

    def _zero(r, _):
        for k in range(SUB // L):
            buf0[r, pl.ds(k * L, L)] = zeros_f
            buf1[r, pl.ds(k * L, L)] = zeros_f
        return _

    lax.fori_loop(0, DEPTH, _zero, None)

    def _block(h, sb, buf, offs, sem, first):
        col = b0 + sb * SUB
        if not first:
            pltpu.make_async_copy(
                buf, out_hbm.at[0, pl.ds(0, DEPTH), pl.ds(0, SUB)], sem
            ).wait()
            for v in range(NVEC):
                c_old = offs[pl.ds(v * L, L)]
                plsc.store_scatter(buf, [c_old, lane + v * L], zeros_f)
        for v in range(NVEC):
            pos = basepos[pl.ds(sb * SUB + v * L, L)] + h
            c = plsc.load_gather(idx_v, [pos])
            offs[pl.ds(v * L, L)] = c
            plsc.store_scatter(buf, [c, lane + v * L], ones_f)
        pltpu.async_copy(
            buf, out_hbm.at[h, pl.ds(0, DEPTH), pl.ds(col, SUB)], sem
        )

    _block(0, 0, buf0, offs0, sem0, True)
    _block(0, 1, buf1, offs1, sem1, True)

    def _step(h, _):
        _block(h, 0, buf0, offs0, sem0, False)
        _block(h, 1, buf1, offs1, sem1, False)
        return _

    lax.fori_loop(1, HIST, _step, None)

    pltpu.make_async_copy(buf0, out_hbm.at[0, pl.ds(0, DEPTH), pl.ds(0, SUB)], sem0).wait()
    pltpu.make_async_copy(buf1, out_hbm.at[0, pl.ds(0, DEPTH), pl.ds(0, SUB)], sem1).wait()


@jax.jit
def _one_hot_sc(flat_idx):
    mesh = plsc.VectorSubcoreMesh(core_axis_name="c", subcore_axis_name="s")
    return pl.kernel(
        _body,
        out_type=jax.ShapeDtypeStruct((HIST, DEPTH, BATCH), jnp.float32),
        mesh=mesh,
        scratch_types=[
            pltpu.VMEM((IDX_PER_W,), jnp.int32),
            pltpu.VMEM((BPW,), jnp.int32),
            pltpu.VMEM((DEPTH, SUB), jnp.float32),
            pltpu.VMEM((DEPTH, SUB), jnp.float32),
            pltpu.VMEM((SUB,), jnp.int32),
            pltpu.VMEM((SUB,), jnp.int32),
            pltpu.SemaphoreType.DMA,
            pltpu.SemaphoreType.DMA,
        ],
        compiler_params=pltpu.CompilerParams(needs_layout_passes=False),
    )(flat_idx)


def kernel(inputs):
    flat = inputs.reshape(BATCH * HIST)
    p = _one_hot_sc(flat)
    return jnp.transpose(p, (2, 0, 1))


# speedup vs baseline: 8.1796x; 1.1041x over previous
import jax
import jax.numpy as jnp
from jax import lax
from jax.experimental import pallas as pl
from jax.experimental.pallas import tpu as pltpu
from jax.experimental.pallas import tpu_sc as plsc

BATCH = 16384
HIST = 50
DEPTH = 101
L = 16
NC, NS = 2, 16
NW = NC * NS
BPW = BATCH // NW
SUB = 256
NSUB = BPW // SUB
NVEC = SUB // L


def _body(in_hbm, out_hbm, idx_v, buf0, buf1, offs0, offs1, sem0, sem1):
    wid = lax.axis_index("s") * NC + lax.axis_index("c")
    b0 = wid * BPW

    pltpu.sync_copy(in_hbm.at[:, pl.ds(b0, BPW)], idx_v)

    zeros_f = jnp.zeros((L,), jnp.float32)
    ones_f = jnp.ones((L,), jnp.float32)
    lane = lax.iota(jnp.int32, L)

    def _zero(r, _):
        for k in range(SUB // L):
            buf0[r, pl.ds(k * L, L)] = zeros_f
            buf1[r, pl.ds(k * L, L)] = zeros_f
        return _

    lax.fori_loop(0, DEPTH, _zero, None)

    def _block(h, sb, buf, offs, sem, first):
        if not first:
            pltpu.make_async_copy(
                buf, out_hbm.at[0, pl.ds(0, DEPTH), pl.ds(0, SUB)], sem
            ).wait()
            for v in range(NVEC):
                c_old = offs[pl.ds(v * L, L)]
                plsc.store_scatter(buf, [c_old, lane + v * L], zeros_f)
        for v in range(NVEC):
            c = idx_v[h, pl.ds(sb * SUB + v * L, L)]
            offs[pl.ds(v * L, L)] = c
            plsc.store_scatter(buf, [c, lane + v * L], ones_f)
        pltpu.async_copy(
            buf, out_hbm.at[h, pl.ds(0, DEPTH), pl.ds(b0 + sb * SUB, SUB)], sem
        )

    _block(0, 0, buf0, offs0, sem0, True)
    _block(0, 1, buf1, offs1, sem1, True)

    def _step(h, _):
        _block(h, 0, buf0, offs0, sem0, False)
        _block(h, 1, buf1, offs1, sem1, False)
        return _

    lax.fori_loop(1, HIST, _step, None)

    pltpu.make_async_copy(buf0, out_hbm.at[0, pl.ds(0, DEPTH), pl.ds(0, SUB)], sem0).wait()
    pltpu.make_async_copy(buf1, out_hbm.at[0, pl.ds(0, DEPTH), pl.ds(0, SUB)], sem1).wait()


@jax.jit
def _one_hot_sc(idx_t):
    mesh = plsc.VectorSubcoreMesh(core_axis_name="c", subcore_axis_name="s")
    return pl.kernel(
        _body,
        out_type=jax.ShapeDtypeStruct((HIST, DEPTH, BATCH), jnp.float32),
        mesh=mesh,
        scratch_types=[
            pltpu.VMEM((HIST, BPW), jnp.int32),
            pltpu.VMEM((DEPTH, SUB), jnp.float32),
            pltpu.VMEM((DEPTH, SUB), jnp.float32),
            pltpu.VMEM((SUB,), jnp.int32),
            pltpu.VMEM((SUB,), jnp.int32),
            pltpu.SemaphoreType.DMA,
            pltpu.SemaphoreType.DMA,
        ],
        compiler_params=pltpu.CompilerParams(needs_layout_passes=False),
    )(idx_t)


def kernel(inputs):
    idx_t = jnp.transpose(inputs)
    p = _one_hot_sc(idx_t)
    return jnp.transpose(p, (2, 0, 1))
